# baseline (speedup 1.0000x reference)
import jax
import jax.numpy as jnp
from jax import lax
from jax.experimental import pallas as pl
from jax.experimental.pallas import tpu as pltpu
from jax.experimental.pallas import tpu_sc as plsc

N = 10000
E = 320000
D = 128
NC = 2
NH = N // NC
NT = 16
CHUNK = 128
CPT = 160
SEG = 40
EPT = CPT * CHUNK
E_PAD = NT * EPT
R = 6144
RPT = R // NT
RB = 128
TRASH = R - NH

_f32 = jnp.float32


def _sc_spmm(src, rows, cols):
    mesh = plsc.VectorSubcoreMesh(core_axis_name="c", subcore_axis_name="s")
    out = jax.ShapeDtypeStruct((N, D), _f32)

    def body(src_hbm, rows_hbm, cols_hbm, out_hbm,
             row_v, col_v, gb0, gb1, gb2, gb3, acc,
             gs0, gs1, gs2, gs3, ss0, ss1, ss2, ss3):
        c = lax.axis_index("c")
        t = lax.axis_index("s")
        gbufs = (gb0, gb1, gb2, gb3)
        gsems = (gs0, gs1, gs2, gs3)
        ssems = (ss0, ss1, ss2, ss3)

        @pl.loop(0, RB)
        def _zr(i):
            @pl.loop(0, D, step=16)
            def _zc(k):
                gb0[i, pl.ds(k, 16)] = jnp.zeros((16,), _f32)

        base = t * RPT
        for r in range(RPT // RB):
            pltpu.sync_copy(gb0, acc.at[pl.ds(base + r * RB, RB)])

        plsc.subcore_barrier()

        @pl.loop(0, CPT, step=SEG)
        def _(s0):
            pltpu.sync_copy(rows_hbm.at[c, t, pl.ds(s0, SEG)], row_v)
            pltpu.sync_copy(cols_hbm.at[c, t, pl.ds(s0, SEG)], col_v)

            for b in range(2):
                pltpu.async_copy(
                    src_hbm.at[col_v.at[b]], gbufs[b], gsems[b])

            @pl.loop(0, SEG, step=4)
            def _(j0):
                for u in range(4):
                    j = j0 + u
                    b = u
                    bp2 = (u + 2) % 4
                    pltpu.make_async_copy(
                        src_hbm.at[col_v.at[j]], gbufs[b], gsems[b]).wait()
                    pltpu.async_copy(
                        gbufs[b], acc.at[row_v.at[j]], ssems[b], add=True)

                    @pl.when(j + 2 < SEG)
                    def _():
                        @pl.when(j >= 2)
                        def _():
                            pltpu.make_async_copy(
                                gbufs[bp2], acc.at[row_v.at[j - 2]],
                                ssems[bp2]).wait()

                        pltpu.async_copy(
                            src_hbm.at[col_v.at[j + 2]], gbufs[bp2],
                            gsems[bp2])

            for u in range(4):
                j = SEG - 4 + u
                pltpu.make_async_copy(
                    gbufs[u], acc.at[row_v.at[j]], ssems[u]).wait()

        plsc.subcore_barrier()

        nfull = NH // RPT
        ntail = NH - nfull * RPT

        @pl.when(t < nfull)
        def _():
            for r in range(RPT // RB):
                pltpu.sync_copy(
                    acc.at[pl.ds(base + r * RB, RB)],
                    out_hbm.at[pl.ds(c * NH + base + r * RB, RB)])

        @pl.when(t == nfull)
        def _():
            pltpu.sync_copy(
                acc.at[pl.ds(base, ntail)],
                out_hbm.at[pl.ds(c * NH + base, ntail)])

    scratch = [
        pltpu.VMEM((SEG, CHUNK), jnp.int32),
        pltpu.VMEM((SEG, CHUNK), jnp.int32),
        pltpu.VMEM((CHUNK, D), _f32),
        pltpu.VMEM((CHUNK, D), _f32),
        pltpu.VMEM((CHUNK, D), _f32),
        pltpu.VMEM((CHUNK, D), _f32),
        pltpu.VMEM_SHARED((R, D), _f32),
    ] + [pltpu.SemaphoreType.DMA] * 8

    k = pl.kernel(body, out_type=out, mesh=mesh, scratch_types=scratch)
    return k(src, rows, cols)


BLK = 1000


def _tc_linear(x, h1, h2, w0t, w1t, w2t, b0, b1, b2):
    def body(x_ref, h1_ref, h2_ref, w0_ref, w1_ref, w2_ref,
             b0_ref, b1_ref, b2_ref, o_ref):
        o_ref[:, 0:D] = (
            jnp.dot(x_ref[...], w0_ref[...], preferred_element_type=_f32)
            + b0_ref[...])
        o_ref[:, D:2 * D] = (
            jnp.dot(h1_ref[...], w1_ref[...], preferred_element_type=_f32)
            + b1_ref[...])
        o_ref[:, 2 * D:3 * D] = (
            jnp.dot(h2_ref[...], w2_ref[...], preferred_element_type=_f32)
            + b2_ref[...])

    nspec = pl.BlockSpec((BLK, D), lambda i: (i, 0))
    wspec = pl.BlockSpec((D, D), lambda i: (0, 0))
    bspec = pl.BlockSpec((1, D), lambda i: (0, 0))
    return pl.pallas_call(
        body,
        grid=(N // BLK,),
        in_specs=[nspec, nspec, nspec, wspec, wspec, wspec,
                  bspec, bspec, bspec],
        out_specs=pl.BlockSpec((BLK, 3 * D), lambda i: (i, 0)),
        out_shape=jax.ShapeDtypeStruct((N, 3 * D), _f32),
    )(x, h1, h2, w0t, w1t, w2t, b0, b1, b2)


def kernel(x, edge_index, W0, b0, W1, b1, W2, b2):
    row = edge_index[0]
    col = edge_index[1]
    pad = E_PAD - E
    eidx = jnp.arange(E, dtype=jnp.int32)
    pidx = jnp.arange(pad, dtype=jnp.int32)
    trash = NH + (eidx % TRASH)
    ptrash = NH + (pidx % TRASH)
    pzero = jnp.zeros((pad,), jnp.int32)

    rows_c = []
    cols_c = []
    for c in range(NC):
        owned = (row >= c * NH) & (row < (c + 1) * NH)
        lrow = jnp.where(owned, row - c * NH, trash)
        lcol = jnp.where(owned, col, 0)
        rows_c.append(jnp.concatenate([lrow, ptrash]))
        cols_c.append(jnp.concatenate([lcol, pzero]))
    rows = jnp.stack(rows_c).reshape(NC, NT, CPT, CHUNK)
    cols = jnp.stack(cols_c).reshape(NC, NT, CPT, CHUNK)

    h1 = _sc_spmm(x, rows, cols)
    h2 = _sc_spmm(h1, rows, cols)
    return _tc_linear(
        x, h1, h2, W0.T, W1.T, W2.T,
        b0.reshape(1, D), b1.reshape(1, D), b2.reshape(1, D))


# speedup vs baseline: 16.4819x; 16.4819x over previous
import jax
import jax.numpy as jnp
from jax import lax
from jax.experimental import pallas as pl
from jax.experimental.pallas import tpu as pltpu
from jax.experimental.pallas import tpu_sc as plsc

N = 10000
E = 320000
D = 128
NC = 2
NH = N // NC
NT = 16
CHUNK = 128
CPT = 160
SEG = 40
EPT = CPT * CHUNK
E_PAD = NT * EPT
R = 6144
RPT = R // NT
RB = 128
TRASH = R - NH

_f32 = jnp.float32


def _sc_spmm(src, rows, cols):
    mesh = plsc.VectorSubcoreMesh(core_axis_name="c", subcore_axis_name="s")
    out = jax.ShapeDtypeStruct((N, D), _f32)

    def body(src_hbm, rows_hbm, cols_hbm, out_hbm,
             row_v, col_v, gb0, gb1, gb2, gb3, acc,
             gs0, gs1, gs2, gs3, ss0, ss1, ss2, ss3):
        c = lax.axis_index("c")
        t = lax.axis_index("s")
        gbufs = (gb0, gb1, gb2, gb3)
        gsems = (gs0, gs1, gs2, gs3)
        ssems = (ss0, ss1, ss2, ss3)

        @pl.loop(0, RB)
        def _zr(i):
            @pl.loop(0, D, step=16)
            def _zc(k):
                gb0[i, pl.ds(k, 16)] = jnp.zeros((16,), _f32)

        base = t * RPT
        for r in range(RPT // RB):
            pltpu.sync_copy(gb0, acc.at[pl.ds(base + r * RB, RB)])

        plsc.subcore_barrier()

        @pl.loop(0, CPT, step=SEG)
        def _(s0):
            pltpu.sync_copy(rows_hbm.at[c, t, pl.ds(s0, SEG)], row_v)
            pltpu.sync_copy(cols_hbm.at[c, t, pl.ds(s0, SEG)], col_v)

            for b in range(2):
                pltpu.async_copy(
                    src_hbm.at[col_v.at[b]], gbufs[b], gsems[b])

            @pl.loop(0, SEG, step=4)
            def _(j0):
                for u in range(4):
                    j = j0 + u
                    b = u
                    bp2 = (u + 2) % 4
                    pltpu.make_async_copy(
                        src_hbm.at[col_v.at[j]], gbufs[b], gsems[b]).wait()
                    pltpu.async_copy(
                        gbufs[b], acc.at[row_v.at[j]], ssems[b], add=True)

                    @pl.when(j + 2 < SEG)
                    def _():
                        @pl.when(j >= 2)
                        def _():
                            pltpu.make_async_copy(
                                gbufs[bp2], acc.at[row_v.at[j - 2]],
                                ssems[bp2]).wait()

                        pltpu.async_copy(
                            src_hbm.at[col_v.at[j + 2]], gbufs[bp2],
                            gsems[bp2])

            for u in range(4):
                j = SEG - 4 + u
                pltpu.make_async_copy(
                    gbufs[u], acc.at[row_v.at[j]], ssems[u]).wait()

        plsc.subcore_barrier()

        nfull = NH // RPT
        ntail = NH - nfull * RPT

        @pl.when(t < nfull)
        def _():
            for r in range(RPT // RB):
                pltpu.sync_copy(
                    acc.at[pl.ds(base + r * RB, RB)],
                    out_hbm.at[pl.ds(c * NH + base + r * RB, RB)])

        @pl.when(t == nfull)
        def _():
            pltpu.sync_copy(
                acc.at[pl.ds(base, ntail)],
                out_hbm.at[pl.ds(c * NH + base, ntail)])

    scratch = [
        pltpu.VMEM((SEG, CHUNK), jnp.int32),
        pltpu.VMEM((SEG, CHUNK), jnp.int32),
        pltpu.VMEM((CHUNK, D), _f32),
        pltpu.VMEM((CHUNK, D), _f32),
        pltpu.VMEM((CHUNK, D), _f32),
        pltpu.VMEM((CHUNK, D), _f32),
        pltpu.VMEM_SHARED((R, D), _f32),
    ] + [pltpu.SemaphoreType.DMA] * 8

    k = pl.kernel(body, out_type=out, mesh=mesh, scratch_types=scratch)
    return k(src, rows, cols)


BLK = 1000


def _tc_linear(x, h1, h2, w0t, w1t, w2t, b0, b1, b2):
    def body(x_ref, h1_ref, h2_ref, w0_ref, w1_ref, w2_ref,
             b0_ref, b1_ref, b2_ref, o_ref):
        o_ref[:, 0:D] = (
            jnp.dot(x_ref[...], w0_ref[...], preferred_element_type=_f32)
            + b0_ref[...])
        o_ref[:, D:2 * D] = (
            jnp.dot(h1_ref[...], w1_ref[...], preferred_element_type=_f32)
            + b1_ref[...])
        o_ref[:, 2 * D:3 * D] = (
            jnp.dot(h2_ref[...], w2_ref[...], preferred_element_type=_f32)
            + b2_ref[...])

    nspec = pl.BlockSpec((BLK, D), lambda i: (i, 0))
    wspec = pl.BlockSpec((D, D), lambda i: (0, 0))
    bspec = pl.BlockSpec((1, D), lambda i: (0, 0))
    return pl.pallas_call(
        body,
        grid=(N // BLK,),
        in_specs=[nspec, nspec, nspec, wspec, wspec, wspec,
                  bspec, bspec, bspec],
        out_specs=pl.BlockSpec((BLK, 3 * D), lambda i: (i, 0)),
        out_shape=jax.ShapeDtypeStruct((N, 3 * D), _f32),
    )(x, h1, h2, w0t, w1t, w2t, b0, b1, b2)


def kernel(x, edge_index, W0, b0, W1, b1, W2, b2):
    row = edge_index[0]
    col = edge_index[1]
    pad = E_PAD - E
    eidx = jnp.arange(E, dtype=jnp.int32)
    pidx = jnp.arange(pad, dtype=jnp.int32)
    trash = NH + (eidx % TRASH)
    ptrash = NH + (pidx % TRASH)
    pzero = jnp.zeros((pad,), jnp.int32)

    rows_c = []
    cols_c = []
    for c in range(NC):
        owned = (row >= c * NH) & (row < (c + 1) * NH)
        lrow = jnp.where(owned, row - c * NH, trash)
        rows_c.append(jnp.concatenate([lrow, ptrash]))
        cols_c.append(jnp.concatenate([col, pzero]))
    rows = jnp.stack(rows_c).reshape(NC, NT, CPT, CHUNK)
    cols = jnp.stack(cols_c).reshape(NC, NT, CPT, CHUNK)

    h1 = _sc_spmm(x, rows, cols)
    h2 = _sc_spmm(h1, rows, cols)
    return _tc_linear(
        x, h1, h2, W0.T, W1.T, W2.T,
        b0.reshape(1, D), b1.reshape(1, D), b2.reshape(1, D))


# speedup vs baseline: 91.2285x; 5.5351x over previous
import jax
import jax.numpy as jnp
from jax import lax
from jax.experimental import pallas as pl
from jax.experimental.pallas import tpu as pltpu
from jax.experimental.pallas import tpu_sc as plsc

N = 10000
E = 320000
D = 128
NC = 2
E2 = E // NC
NT = 16
CHUNK = 128
CPT = 80
SEG = 40
EPT = CPT * CHUNK
E_PAD = NT * EPT
R = 10240
RPT = R // NT
RB = 128
TRASH = R - N

_f32 = jnp.float32


def _sc_spmm(src, rows, cols):
    mesh = plsc.VectorSubcoreMesh(core_axis_name="c", subcore_axis_name="s")
    out = jax.ShapeDtypeStruct((NC, N, D), _f32)

    def body(src_hbm, rows_hbm, cols_hbm, out_hbm,
             row_v, col_v, gb0, gb1, acc,
             gs0, gs1, ss0, ss1):
        c = lax.axis_index("c")
        t = lax.axis_index("s")
        gbufs = (gb0, gb1)
        gsems = (gs0, gs1)
        ssems = (ss0, ss1)

        @pl.loop(0, RB)
        def _zr(i):
            @pl.loop(0, D, step=16)
            def _zc(k):
                gb0[i, pl.ds(k, 16)] = jnp.zeros((16,), _f32)

        base = t * RPT
        for r in range(RPT // RB):
            pltpu.sync_copy(gb0, acc.at[pl.ds(base + r * RB, RB)])

        plsc.subcore_barrier()

        @pl.loop(0, CPT, step=SEG)
        def _(s0):
            pltpu.sync_copy(rows_hbm.at[c, t, pl.ds(s0, SEG)], row_v)
            pltpu.sync_copy(cols_hbm.at[c, t, pl.ds(s0, SEG)], col_v)

            for b in range(2):
                pltpu.async_copy(
                    src_hbm.at[col_v.at[b]], gbufs[b], gsems[b])

            @pl.loop(0, SEG, step=2)
            def _(j0):
                for u in range(2):
                    j = j0 + u
                    b = u
                    pltpu.make_async_copy(
                        src_hbm.at[col_v.at[j]], gbufs[b], gsems[b]).wait()
                    pltpu.async_copy(
                        gbufs[b], acc.at[row_v.at[j]], ssems[b], add=True)
                    pltpu.make_async_copy(
                        gbufs[b], acc.at[row_v.at[j]], ssems[b]).wait()

                    @pl.when(j + 2 < SEG)
                    def _():
                        pltpu.async_copy(
                            src_hbm.at[col_v.at[j + 2]], gbufs[b], gsems[b])

        plsc.subcore_barrier()

        nfull = N // RPT
        ntail = N - nfull * RPT

        @pl.when(t < nfull)
        def _():
            pltpu.sync_copy(
                acc.at[pl.ds(base, RPT)], out_hbm.at[c, pl.ds(base, RPT)])

        @pl.when(t == nfull)
        def _():
            pltpu.sync_copy(
                acc.at[pl.ds(base, ntail)],
                out_hbm.at[c, pl.ds(base, ntail)])

    scratch = [
        pltpu.VMEM((SEG, CHUNK), jnp.int32),
        pltpu.VMEM((SEG, CHUNK), jnp.int32),
        pltpu.VMEM((CHUNK, D), _f32),
        pltpu.VMEM((CHUNK, D), _f32),
        pltpu.VMEM_SHARED((R, D), _f32),
    ] + [pltpu.SemaphoreType.DMA] * 4

    k = pl.kernel(body, out_type=out, mesh=mesh, scratch_types=scratch)
    return k(src, rows, cols)


BLK = 1000


def _tc_add(p):
    def body(p_ref, o_ref):
        o_ref[...] = p_ref[0] + p_ref[1]

    return pl.pallas_call(
        body,
        grid=(N // BLK,),
        in_specs=[pl.BlockSpec((NC, BLK, D), lambda i: (0, i, 0))],
        out_specs=pl.BlockSpec((BLK, D), lambda i: (i, 0)),
        out_shape=jax.ShapeDtypeStruct((N, D), _f32),
    )(p)


def _tc_linear(x, h1, p2, w0t, w1t, w2t, b0, b1, b2):
    def body(x_ref, h1_ref, p2_ref, w0_ref, w1_ref, w2_ref,
             b0_ref, b1_ref, b2_ref, o_ref):
        o_ref[:, 0:D] = (
            jnp.dot(x_ref[...], w0_ref[...], preferred_element_type=_f32)
            + b0_ref[...])
        o_ref[:, D:2 * D] = (
            jnp.dot(h1_ref[...], w1_ref[...], preferred_element_type=_f32)
            + b1_ref[...])
        h2 = p2_ref[0] + p2_ref[1]
        o_ref[:, 2 * D:3 * D] = (
            jnp.dot(h2, w2_ref[...], preferred_element_type=_f32)
            + b2_ref[...])

    nspec = pl.BlockSpec((BLK, D), lambda i: (i, 0))
    pspec = pl.BlockSpec((NC, BLK, D), lambda i: (0, i, 0))
    wspec = pl.BlockSpec((D, D), lambda i: (0, 0))
    bspec = pl.BlockSpec((1, D), lambda i: (0, 0))
    return pl.pallas_call(
        body,
        grid=(N // BLK,),
        in_specs=[nspec, nspec, pspec, wspec, wspec, wspec,
                  bspec, bspec, bspec],
        out_specs=pl.BlockSpec((BLK, 3 * D), lambda i: (i, 0)),
        out_shape=jax.ShapeDtypeStruct((N, 3 * D), _f32),
    )(x, h1, p2, w0t, w1t, w2t, b0, b1, b2)


def kernel(x, edge_index, W0, b0, W1, b1, W2, b2):
    row = edge_index[0]
    col = edge_index[1]
    pad = E_PAD - E2
    pidx = jnp.arange(pad, dtype=jnp.int32)
    ptrash = N + (pidx % TRASH)
    pcol = pidx % N

    rows_c = []
    cols_c = []
    for c in range(NC):
        sl = slice(c * E2, (c + 1) * E2)
        rows_c.append(jnp.concatenate([row[sl], ptrash]))
        cols_c.append(jnp.concatenate([col[sl], pcol]))
    rows = jnp.stack(rows_c).reshape(NC, NT, CPT, CHUNK)
    cols = jnp.stack(cols_c).reshape(NC, NT, CPT, CHUNK)

    p1 = _sc_spmm(x, rows, cols)
    h1 = _tc_add(p1)
    p2 = _sc_spmm(h1, rows, cols)
    return _tc_linear(
        x, h1, p2, W0.T, W1.T, W2.T,
        b0.reshape(1, D), b1.reshape(1, D), b2.reshape(1, D))
